# baseline (speedup 1.0000x reference)
import functools

import jax
import jax.numpy as jnp
from jax import lax
from jax.experimental import pallas as pl
from jax.experimental.pallas import tpu as pltpu
from jax.experimental.pallas import tpu_sc as plsc

D_MODEL = 512
BUFFER_SIZE = 4096
MMAP_SIZE = 100000

_NC = 2
_NS = 16
_NW = _NC * _NS
_RPO = MMAP_SIZE // _NW
_TBL = 3136
_ECAP = 3328
_LCAP = 4224
_NEV = BUFFER_SIZE // 16
_CHUNK = 128

_mesh = plsc.VectorSubcoreMesh(core_axis_name="c", subcore_axis_name="s")


@functools.partial(
    pl.kernel,
    out_type=jax.ShapeDtypeStruct((BUFFER_SIZE, D_MODEL), jnp.float32),
    mesh=_mesh,
    scratch_types=[
        pltpu.VMEM((BUFFER_SIZE,), jnp.int32),
        pltpu.VMEM((BUFFER_SIZE,), jnp.int32),
        pltpu.VMEM((_TBL,), jnp.int32),
        pltpu.VMEM((_ECAP,), jnp.int32),
        pltpu.VMEM((_ECAP,), jnp.int32),
        pltpu.VMEM((_LCAP,), jnp.int32),
        pltpu.VMEM((_LCAP,), jnp.int32),
        pltpu.VMEM((_CHUNK, D_MODEL), jnp.float32),
        pltpu.SemaphoreType.DMA,
    ],
    compiler_params=pltpu.CompilerParams(needs_layout_passes=False),
)
def _sc_main(m_ref, dbuf_hbm, evict_hbm, load_hbm, out_hbm,
             ev_v, ld_v, tbl, ec_v, wsrc_v, lc_v, lpos_v, rows_v, sem):
    wid = lax.axis_index("s") * _NC + lax.axis_index("c")
    lo = wid * _RPO
    hi = lo + _RPO
    iota = lax.broadcasted_iota(jnp.int32, (16,), 0)

    pltpu.sync_copy(evict_hbm, ev_v)
    pltpu.sync_copy(load_hbm, ld_v)

    def init(t, c):
        tbl[pl.ds(16 * t, 16)] = jnp.full((16,), -1, jnp.int32)
        return c

    lax.fori_loop(0, _TBL // 16, init, 0)

    def build(q, c):
        ev = ev_v[pl.ds(16 * q, 16)]
        mine = (ev >= lo) & (ev < hi)
        _, last = plsc.scan_count(ev, mask=mine)
        idx = jnp.where(mine, ev - lo, 0)
        plsc.store_scatter(tbl, (idx,), 16 * q + iota, mask=last & mine)
        return c

    lax.fori_loop(0, _NEV, build, 0)

    def comp_e(t, n):
        vals = tbl[pl.ds(16 * t, 16)]
        m = vals >= 0
        plsc.store_compressed(ec_v.at[pl.ds(n, 16)], lo + 16 * t + iota, mask=m)
        plsc.store_compressed(wsrc_v.at[pl.ds(n, 16)], vals, mask=m)
        return n + jnp.max(plsc.all_reduce_population_count(m))

    n_e = lax.fori_loop(0, _TBL // 16, comp_e, 0)

    def comp_l(q, n):
        ldq = ld_v[pl.ds(16 * q, 16)]
        m = (ldq >= lo) & (ldq < hi)
        plsc.store_compressed(lc_v.at[pl.ds(n, 16)], ldq, mask=m)
        plsc.store_compressed(lpos_v.at[pl.ds(n, 16)], 16 * q + iota, mask=m)
        return n + jnp.max(plsc.all_reduce_population_count(m))

    n_l = lax.fori_loop(0, _NEV, comp_l, 0)

    def pad_lists(n, a_v, b_v):
        a0 = a_v[pl.ds(0, 16)]
        b0 = b_v[pl.ds(0, 16)]
        zer = jnp.zeros((16,), jnp.int32)
        a0b = a0.at[zer].get(mode="promise_in_bounds")
        b0b = b0.at[zer].get(mode="promise_in_bounds")
        base8 = (n // 8) * 8

        def pad(f, c):
            off = base8 + 8 * f
            m = (off + iota) >= n
            a_v[pl.ds(off, 16)] = jnp.where(m, a0b, a_v[pl.ds(off, 16)])
            b_v[pl.ds(off, 16)] = jnp.where(m, b0b, b_v[pl.ds(off, 16)])
            return c

        lax.fori_loop(0, 16, pad, 0)

    @pl.when(n_e > 0)
    def _():
        pad_lists(n_e, ec_v, wsrc_v)

        def chunk_e(c, carry):
            off = c * _CHUNK
            pltpu.async_copy(
                dbuf_hbm.at[wsrc_v.at[pl.ds(off, _CHUNK)]], rows_v, sem
            ).wait()
            pltpu.async_copy(
                rows_v, m_ref.at[ec_v.at[pl.ds(off, _CHUNK)]], sem
            ).wait()
            return carry

        lax.fori_loop(0, (n_e + _CHUNK - 1) // _CHUNK, chunk_e, 0)

    @pl.when(n_l > 0)
    def _():
        pad_lists(n_l, lc_v, lpos_v)

        def chunk_l(c, carry):
            off = c * _CHUNK
            pltpu.async_copy(
                m_ref.at[lc_v.at[pl.ds(off, _CHUNK)]], rows_v, sem
            ).wait()
            pltpu.async_copy(
                rows_v, out_hbm.at[lpos_v.at[pl.ds(off, _CHUNK)]], sem
            ).wait()
            return carry

        lax.fori_loop(0, (n_l + _CHUNK - 1) // _CHUNK, chunk_l, 0)


def kernel(mmap, device_buffer, load_indices, evict_indices):
    evict = evict_indices.astype(jnp.int32)
    load = load_indices.astype(jnp.int32)
    m_ref = jax.new_ref(mmap)
    new_buffer = _sc_main(m_ref, device_buffer, evict, load)
    new_mmap = jax.freeze(m_ref)
    return (new_buffer, new_mmap)
